# Initial kernel scaffold; baseline (speedup 1.0000x reference)
import jax
import jax.numpy as jnp
from jax.experimental import pallas as pl


def kernel(x, edge_index, W0, b0, W1, b1, W2, b2):
    raise NotImplementedError("write your pallas kernel here")


# speedup vs baseline: 18.6636x; 18.6636x over previous
import functools

import jax
import jax.numpy as jnp
from jax import lax
from jax.experimental import pallas as pl
from jax.experimental.pallas import tpu as pltpu
from jax.experimental.pallas import tpu_sc as plsc

NC = 2
NS = 16
NW = NC * NS
C = 128
GRP = 16


def _fill(ref, n, value):
    def body(i, _):
        ref[pl.ds(i * 16, 16)] = jnp.full((16,), value, jnp.float32)
        return 0
    lax.fori_loop(0, n // 16, body, 0)


def _fill2d(ref, rows, cols, value):
    def body(i, _):
        r = i // (cols // 16)
        c = i % (cols // 16)
        ref[r, pl.ds(c * 16, 16)] = jnp.full((16,), value, jnp.float32)
        return 0
    lax.fori_loop(0, rows * (cols // 16), body, 0)


def _sc_degree(dst3d, np_rows, nch):
    rows_per_tile = np_rows // NS
    mesh = plsc.VectorSubcoreMesh(core_axis_name="c", subcore_axis_name="s")

    @functools.partial(
        pl.kernel,
        out_type=jax.ShapeDtypeStruct((NC, np_rows), jnp.float32),
        mesh=mesh,
        scratch_types=[
            pltpu.VMEM_SHARED((np_rows,), jnp.float32),
            pltpu.VMEM((GRP, C), jnp.int32),
            pltpu.VMEM((C,), jnp.float32),
            pltpu.VMEM((rows_per_tile,), jnp.float32),
        ],
    )
    def deg_kernel(dst_hbm, out_hbm, acc, dst_v, ones_v, zeros_v):
        cid = lax.axis_index("c")
        sid = lax.axis_index("s")
        wid = cid * NS + sid
        _fill(ones_v, C, 1.0)
        _fill(zeros_v, rows_per_tile, 0.0)
        pltpu.sync_copy(zeros_v, acc.at[pl.ds(sid * rows_per_tile, rows_per_tile)])
        plsc.subcore_barrier()

        def group(gi, _):
            pltpu.sync_copy(dst_hbm.at[wid, pl.ds(gi * GRP, GRP)], dst_v)

            def chunk(j, _):
                pltpu.sync_copy(ones_v, acc.at[dst_v.at[j]], add=True)
                return 0
            lax.fori_loop(0, GRP, chunk, 0)
            return 0
        lax.fori_loop(0, nch // GRP, group, 0)
        plsc.subcore_barrier()
        pltpu.sync_copy(acc.at[pl.ds(sid * rows_per_tile, rows_per_tile)],
                        out_hbm.at[cid, pl.ds(sid * rows_per_tile, rows_per_tile)])

    return deg_kernel(dst3d)


def _sc_aggregate(g, src3d, dst3d, np_rows, nch, d):
    rows_per_tile = np_rows // NS
    mesh = plsc.VectorSubcoreMesh(core_axis_name="c", subcore_axis_name="s")

    @functools.partial(
        pl.kernel,
        out_type=jax.ShapeDtypeStruct((NC, np_rows, d), jnp.float32),
        mesh=mesh,
        scratch_types=[
            pltpu.VMEM_SHARED((np_rows, d), jnp.float32),
            pltpu.VMEM((GRP, C), jnp.int32),
            pltpu.VMEM((GRP, C), jnp.int32),
            pltpu.VMEM((2, C, d), jnp.float32),
            pltpu.SemaphoreType.DMA,
            pltpu.SemaphoreType.DMA,
        ],
    )
    def agg_kernel(g_hbm, src_hbm, dst_hbm, out_hbm,
                   acc, src_v, dst_v, rows_v, sem0, sem1):
        cid = lax.axis_index("c")
        sid = lax.axis_index("s")
        wid = cid * NS + sid
        _fill2d(rows_v.at[0], C, d, 0.0)
        for k in range(rows_per_tile // C):
            pltpu.sync_copy(rows_v.at[0],
                            acc.at[pl.ds(sid * rows_per_tile + k * C, C)])
        plsc.subcore_barrier()

        def group(gi, _):
            pltpu.sync_copy(src_hbm.at[wid, pl.ds(gi * GRP, GRP)], src_v)
            pltpu.sync_copy(dst_hbm.at[wid, pl.ds(gi * GRP, GRP)], dst_v)

            def chunk(j, _):
                pltpu.async_copy(g_hbm.at[src_v.at[j]], rows_v.at[0], sem0).wait()
                pltpu.sync_copy(rows_v.at[0], acc.at[dst_v.at[j]], add=True)
                return 0
            lax.fori_loop(0, GRP, chunk, 0)
            return 0
        lax.fori_loop(0, nch // GRP, group, 0)
        plsc.subcore_barrier()
        pltpu.sync_copy(acc.at[pl.ds(sid * rows_per_tile, rows_per_tile)],
                        out_hbm.at[cid, pl.ds(sid * rows_per_tile, rows_per_tile)])

    return agg_kernel(g, src3d, dst3d)


def _tc_first(degp, xp, w0, np_rows, blk):
    din, dh = w0.shape

    def body(degp_ref, x_ref, w_ref, dinv_ref, g_ref):
        deg = degp_ref[0] + degp_ref[1] + 1.0
        dv = lax.rsqrt(deg)
        dinv_ref[...] = dv
        h = jnp.dot(x_ref[...], w_ref[...], preferred_element_type=jnp.float32)
        g_ref[...] = h * dv[:, None]

    grid = (np_rows // blk,)
    return pl.pallas_call(
        body,
        grid=grid,
        in_specs=[
            pl.BlockSpec((NC, blk), lambda i: (0, i)),
            pl.BlockSpec((blk, din), lambda i: (i, 0)),
            pl.BlockSpec((din, dh), lambda i: (0, 0)),
        ],
        out_specs=[
            pl.BlockSpec((blk,), lambda i: (i,)),
            pl.BlockSpec((blk, dh), lambda i: (i, 0)),
        ],
        out_shape=[
            jax.ShapeDtypeStruct((np_rows,), jnp.float32),
            jax.ShapeDtypeStruct((np_rows, dh), jnp.float32),
        ],
    )(degp, xp, w0)


def _tc_mid(aggp, g, dinv, b, w, np_rows, blk):
    d, dn = w.shape

    def body(aggp_ref, g_ref, dinv_ref, b_ref, w_ref, out_ref):
        s = aggp_ref[0] + aggp_ref[1] + g_ref[...]
        dv = dinv_ref[...]
        h = jnp.maximum(s * dv[:, None] + b_ref[...][None, :], 0.0)
        out_ref[...] = jnp.dot(h, w_ref[...],
                               preferred_element_type=jnp.float32) * dv[:, None]

    grid = (np_rows // blk,)
    return pl.pallas_call(
        body,
        grid=grid,
        in_specs=[
            pl.BlockSpec((NC, blk, d), lambda i: (0, i, 0)),
            pl.BlockSpec((blk, d), lambda i: (i, 0)),
            pl.BlockSpec((blk,), lambda i: (i,)),
            pl.BlockSpec((d,), lambda i: (0,)),
            pl.BlockSpec((d, dn), lambda i: (0, 0)),
        ],
        out_specs=pl.BlockSpec((blk, dn), lambda i: (i, 0)),
        out_shape=jax.ShapeDtypeStruct((np_rows, dn), jnp.float32),
    )(aggp, g, dinv, b, w)


def _tc_final(aggp, g, dinv, b, np_rows, blk):
    d = g.shape[1]
    dout = b.shape[0]

    def body(aggp_ref, g_ref, dinv_ref, b_ref, out_ref):
        s = aggp_ref[0] + aggp_ref[1] + g_ref[...]
        v = (s * dinv_ref[...][:, None])[:, :dout] + b_ref[...][None, :]
        m = jnp.max(v, axis=-1, keepdims=True)
        e = v - m
        out_ref[...] = e - jnp.log(jnp.sum(jnp.exp(e), axis=-1, keepdims=True))

    grid = (np_rows // blk,)
    return pl.pallas_call(
        body,
        grid=grid,
        in_specs=[
            pl.BlockSpec((NC, blk, d), lambda i: (0, i, 0)),
            pl.BlockSpec((blk, d), lambda i: (i, 0)),
            pl.BlockSpec((blk,), lambda i: (i,)),
            pl.BlockSpec((dout,), lambda i: (0,)),
        ],
        out_specs=pl.BlockSpec((blk, dout), lambda i: (i, 0)),
        out_shape=jax.ShapeDtypeStruct((np_rows, dout), jnp.float32),
    )(aggp, g, dinv, b)


def kernel(x, edge_index, W0, b0, W1, b1, W2, b2):
    n, din = x.shape
    e = edge_index.shape[1]

    np_rows = ((n + 16) + 16 * NW - 1) // (16 * NW) * (16 * NW)
    n_pad_rows = np_rows - n
    ept = (e + NW - 1) // NW
    nch = (ept + C * GRP - 1) // (C * GRP) * GRP
    e_pad = NW * nch * C
    pad = e_pad - e

    pad_ids = n + (jnp.arange(pad, dtype=jnp.int32) % n_pad_rows)
    src3d = jnp.concatenate([edge_index[0], pad_ids]).reshape(NW, nch, C)
    dst3d = jnp.concatenate([edge_index[1], pad_ids]).reshape(NW, nch, C)
    xp = jnp.pad(x, ((0, np_rows - n), (0, 0)))

    blk = 1024
    degp = _sc_degree(dst3d, np_rows, nch)
    dinv, g0 = _tc_first(degp, xp, W0, np_rows, blk)
    a0 = _sc_aggregate(g0, src3d, dst3d, np_rows, nch, W0.shape[1])
    g1 = _tc_mid(a0, g0, dinv, b0, W1, np_rows, blk)
    a1 = _sc_aggregate(g1, src3d, dst3d, np_rows, nch, W1.shape[1])
    W2p = jnp.pad(W2, ((0, 0), (0, 128 - W2.shape[1])))
    g2 = _tc_mid(a1, g1, dinv, b1, W2p, np_rows, blk)
    a2 = _sc_aggregate(g2, src3d, dst3d, np_rows, nch, W2p.shape[1])
    out = _tc_final(a2, g2, dinv, b2, np_rows, blk)
    return out[:n]


# speedup vs baseline: 26.3264x; 1.4106x over previous
import functools

import jax
import jax.numpy as jnp
from jax import lax
from jax.experimental import pallas as pl
from jax.experimental.pallas import tpu as pltpu
from jax.experimental.pallas import tpu_sc as plsc

NC = 2
NS = 16
NW = NC * NS
C = 128
GRP = 16


def _fill(ref, n, value):
    def body(i, _):
        ref[pl.ds(i * 16, 16)] = jnp.full((16,), value, jnp.float32)
        return 0
    lax.fori_loop(0, n // 16, body, 0)


def _fill2d(ref, rows, cols, value):
    def body(i, _):
        r = i // (cols // 16)
        c = i % (cols // 16)
        ref[r, pl.ds(c * 16, 16)] = jnp.full((16,), value, jnp.float32)
        return 0
    lax.fori_loop(0, rows * (cols // 16), body, 0)


def _sc_degree(dst3d, np_rows, nch):
    rows_per_tile = np_rows // NS
    mesh = plsc.VectorSubcoreMesh(core_axis_name="c", subcore_axis_name="s")

    @functools.partial(
        pl.kernel,
        out_type=jax.ShapeDtypeStruct((NC, np_rows), jnp.float32),
        mesh=mesh,
        scratch_types=[
            pltpu.VMEM_SHARED((np_rows,), jnp.float32),
            pltpu.VMEM((GRP, C), jnp.int32),
            pltpu.VMEM((C,), jnp.float32),
            pltpu.VMEM((rows_per_tile,), jnp.float32),
        ],
    )
    def deg_kernel(dst_hbm, out_hbm, acc, dst_v, ones_v, zeros_v):
        cid = lax.axis_index("c")
        sid = lax.axis_index("s")
        wid = cid * NS + sid
        _fill(ones_v, C, 1.0)
        _fill(zeros_v, rows_per_tile, 0.0)
        pltpu.sync_copy(zeros_v, acc.at[pl.ds(sid * rows_per_tile, rows_per_tile)])
        plsc.subcore_barrier()

        def group(gi, _):
            pltpu.sync_copy(dst_hbm.at[wid, pl.ds(gi * GRP, GRP)], dst_v)

            def chunk(j, _):
                pltpu.sync_copy(ones_v, acc.at[dst_v.at[j]], add=True)
                return 0
            lax.fori_loop(0, GRP, chunk, 0)
            return 0
        lax.fori_loop(0, nch // GRP, group, 0)
        plsc.subcore_barrier()
        pltpu.sync_copy(acc.at[pl.ds(sid * rows_per_tile, rows_per_tile)],
                        out_hbm.at[cid, pl.ds(sid * rows_per_tile, rows_per_tile)])

    return deg_kernel(dst3d)


def _sc_aggregate(g, src3d, dst3d, np_rows, nch, d):
    rows_per_tile = np_rows // NS
    mesh = plsc.VectorSubcoreMesh(core_axis_name="c", subcore_axis_name="s")

    @functools.partial(
        pl.kernel,
        out_type=jax.ShapeDtypeStruct((NC, np_rows, d), jnp.float32),
        mesh=mesh,
        scratch_types=[
            pltpu.VMEM_SHARED((np_rows, d), jnp.float32),
            pltpu.VMEM((GRP, C), jnp.int32),
            pltpu.VMEM((GRP, C), jnp.int32),
            pltpu.VMEM((2, C, d), jnp.float32),
            pltpu.SemaphoreType.DMA,
            pltpu.SemaphoreType.DMA,
        ],
    )
    def agg_kernel(g_hbm, src_hbm, dst_hbm, out_hbm,
                   acc, src_v, dst_v, rows_v, sem0, sem1):
        cid = lax.axis_index("c")
        sid = lax.axis_index("s")
        wid = cid * NS + sid
        _fill2d(rows_v.at[0], C, d, 0.0)
        for k in range(rows_per_tile // C):
            pltpu.sync_copy(rows_v.at[0],
                            acc.at[pl.ds(sid * rows_per_tile + k * C, C)])
        plsc.subcore_barrier()

        def wait_gather(buf, sem):
            pltpu.make_async_copy(g_hbm.at[pl.ds(0, C)], buf, sem).wait()

        def group(gi, _):
            pltpu.sync_copy(src_hbm.at[wid, pl.ds(gi * GRP, GRP)], src_v)
            pltpu.sync_copy(dst_hbm.at[wid, pl.ds(gi * GRP, GRP)], dst_v)
            pltpu.async_copy(g_hbm.at[src_v.at[0]], rows_v.at[0], sem0)

            def pair(t, _):
                pltpu.async_copy(g_hbm.at[src_v.at[2 * t + 1]], rows_v.at[1],
                                 sem1)
                wait_gather(rows_v.at[0], sem0)
                pltpu.sync_copy(rows_v.at[0], acc.at[dst_v.at[2 * t]],
                                add=True)

                @pl.when(t + 1 < GRP // 2)
                def _():
                    pltpu.async_copy(g_hbm.at[src_v.at[2 * t + 2]],
                                     rows_v.at[0], sem0)
                wait_gather(rows_v.at[1], sem1)
                pltpu.sync_copy(rows_v.at[1], acc.at[dst_v.at[2 * t + 1]],
                                add=True)
                return 0
            lax.fori_loop(0, GRP // 2, pair, 0)
            return 0
        lax.fori_loop(0, nch // GRP, group, 0)
        plsc.subcore_barrier()
        pltpu.sync_copy(acc.at[pl.ds(sid * rows_per_tile, rows_per_tile)],
                        out_hbm.at[cid, pl.ds(sid * rows_per_tile, rows_per_tile)])

    return agg_kernel(g, src3d, dst3d)


def _tc_first(degp, xp, w0, np_rows, blk):
    din, dh = w0.shape

    def body(degp_ref, x_ref, w_ref, dinv_ref, g_ref):
        deg = degp_ref[0] + degp_ref[1] + 1.0
        dv = lax.rsqrt(deg)
        dinv_ref[...] = dv
        h = jnp.dot(x_ref[...], w_ref[...], preferred_element_type=jnp.float32)
        g_ref[...] = h * dv[:, None]

    grid = (np_rows // blk,)
    return pl.pallas_call(
        body,
        grid=grid,
        in_specs=[
            pl.BlockSpec((NC, blk), lambda i: (0, i)),
            pl.BlockSpec((blk, din), lambda i: (i, 0)),
            pl.BlockSpec((din, dh), lambda i: (0, 0)),
        ],
        out_specs=[
            pl.BlockSpec((blk,), lambda i: (i,)),
            pl.BlockSpec((blk, dh), lambda i: (i, 0)),
        ],
        out_shape=[
            jax.ShapeDtypeStruct((np_rows,), jnp.float32),
            jax.ShapeDtypeStruct((np_rows, dh), jnp.float32),
        ],
    )(degp, xp, w0)


def _tc_mid(aggp, g, dinv, b, w, np_rows, blk):
    d, dn = w.shape

    def body(aggp_ref, g_ref, dinv_ref, b_ref, w_ref, out_ref):
        s = aggp_ref[0] + aggp_ref[1] + g_ref[...]
        dv = dinv_ref[...]
        h = jnp.maximum(s * dv[:, None] + b_ref[...][None, :], 0.0)
        out_ref[...] = jnp.dot(h, w_ref[...],
                               preferred_element_type=jnp.float32) * dv[:, None]

    grid = (np_rows // blk,)
    return pl.pallas_call(
        body,
        grid=grid,
        in_specs=[
            pl.BlockSpec((NC, blk, d), lambda i: (0, i, 0)),
            pl.BlockSpec((blk, d), lambda i: (i, 0)),
            pl.BlockSpec((blk,), lambda i: (i,)),
            pl.BlockSpec((d,), lambda i: (0,)),
            pl.BlockSpec((d, dn), lambda i: (0, 0)),
        ],
        out_specs=pl.BlockSpec((blk, dn), lambda i: (i, 0)),
        out_shape=jax.ShapeDtypeStruct((np_rows, dn), jnp.float32),
    )(aggp, g, dinv, b, w)


def _tc_final(aggp, g, dinv, b, np_rows, blk):
    d = g.shape[1]
    dout = b.shape[0]

    def body(aggp_ref, g_ref, dinv_ref, b_ref, out_ref):
        s = aggp_ref[0] + aggp_ref[1] + g_ref[...]
        v = (s * dinv_ref[...][:, None])[:, :dout] + b_ref[...][None, :]
        m = jnp.max(v, axis=-1, keepdims=True)
        e = v - m
        out_ref[...] = e - jnp.log(jnp.sum(jnp.exp(e), axis=-1, keepdims=True))

    grid = (np_rows // blk,)
    return pl.pallas_call(
        body,
        grid=grid,
        in_specs=[
            pl.BlockSpec((NC, blk, d), lambda i: (0, i, 0)),
            pl.BlockSpec((blk, d), lambda i: (i, 0)),
            pl.BlockSpec((blk,), lambda i: (i,)),
            pl.BlockSpec((dout,), lambda i: (0,)),
        ],
        out_specs=pl.BlockSpec((blk, dout), lambda i: (i, 0)),
        out_shape=jax.ShapeDtypeStruct((np_rows, dout), jnp.float32),
    )(aggp, g, dinv, b)


def kernel(x, edge_index, W0, b0, W1, b1, W2, b2):
    n, din = x.shape
    e = edge_index.shape[1]

    np_rows = ((n + 16) + 16 * NW - 1) // (16 * NW) * (16 * NW)
    n_pad_rows = np_rows - n
    ept = (e + NW - 1) // NW
    nch = (ept + C * GRP - 1) // (C * GRP) * GRP
    e_pad = NW * nch * C
    pad = e_pad - e

    pad_ids = n + (jnp.arange(pad, dtype=jnp.int32) % n_pad_rows)
    src3d = jnp.concatenate([edge_index[0], pad_ids]).reshape(NW, nch, C)
    dst3d = jnp.concatenate([edge_index[1], pad_ids]).reshape(NW, nch, C)
    xp = jnp.pad(x, ((0, np_rows - n), (0, 0)))

    blk = 1024
    degp = _sc_degree(dst3d, np_rows, nch)
    dinv, g0 = _tc_first(degp, xp, W0, np_rows, blk)
    a0 = _sc_aggregate(g0, src3d, dst3d, np_rows, nch, W0.shape[1])
    g1 = _tc_mid(a0, g0, dinv, b0, W1, np_rows, blk)
    a1 = _sc_aggregate(g1, src3d, dst3d, np_rows, nch, W1.shape[1])
    W2p = jnp.pad(W2, ((0, 0), (0, 128 - W2.shape[1])))
    g2 = _tc_mid(a1, g1, dinv, b1, W2p, np_rows, blk)
    a2 = _sc_aggregate(g2, src3d, dst3d, np_rows, nch, W2p.shape[1])
    out = _tc_final(a2, g2, dinv, b2, np_rows, blk)
    return out[:n]


# speedup vs baseline: 27.2518x; 1.0352x over previous
import functools

import jax
import jax.numpy as jnp
from jax import lax
from jax.experimental import pallas as pl
from jax.experimental.pallas import tpu as pltpu
from jax.experimental.pallas import tpu_sc as plsc

NC = 2
NS = 16
NW = NC * NS
C = 128
GRP = 16


def _fill(ref, n, value):
    def body(i, _):
        ref[pl.ds(i * 16, 16)] = jnp.full((16,), value, jnp.float32)
        return 0
    lax.fori_loop(0, n // 16, body, 0)


def _fill2d(ref, rows, cols, value):
    def body(i, _):
        r = i // (cols // 16)
        c = i % (cols // 16)
        ref[r, pl.ds(c * 16, 16)] = jnp.full((16,), value, jnp.float32)
        return 0
    lax.fori_loop(0, rows * (cols // 16), body, 0)


def _sc_degree(dst3d, np_rows, nch):
    rows_per_tile = np_rows // NS
    mesh = plsc.VectorSubcoreMesh(core_axis_name="c", subcore_axis_name="s")

    @functools.partial(
        pl.kernel,
        out_type=jax.ShapeDtypeStruct((NC, np_rows), jnp.float32),
        mesh=mesh,
        scratch_types=[
            pltpu.VMEM_SHARED((np_rows,), jnp.float32),
            pltpu.VMEM((GRP, C), jnp.int32),
            pltpu.VMEM((C,), jnp.float32),
            pltpu.VMEM((rows_per_tile,), jnp.float32),
        ],
    )
    def deg_kernel(dst_hbm, out_hbm, acc, dst_v, ones_v, zeros_v):
        cid = lax.axis_index("c")
        sid = lax.axis_index("s")
        wid = cid * NS + sid
        _fill(ones_v, C, 1.0)
        _fill(zeros_v, rows_per_tile, 0.0)
        pltpu.sync_copy(zeros_v, acc.at[pl.ds(sid * rows_per_tile, rows_per_tile)])
        plsc.subcore_barrier()

        def group(gi, _):
            pltpu.sync_copy(dst_hbm.at[wid, pl.ds(gi * GRP, GRP)], dst_v)

            def chunk(j, _):
                pltpu.sync_copy(ones_v, acc.at[dst_v.at[j]], add=True)
                return 0
            lax.fori_loop(0, GRP, chunk, 0)
            return 0
        lax.fori_loop(0, nch // GRP, group, 0)
        plsc.subcore_barrier()
        pltpu.sync_copy(acc.at[pl.ds(sid * rows_per_tile, rows_per_tile)],
                        out_hbm.at[cid, pl.ds(sid * rows_per_tile, rows_per_tile)])

    return deg_kernel(dst3d)


def _sc_aggregate(g, src3d, dst3d, np_rows, nch, d):
    rows_per_tile = np_rows // NS
    mesh = plsc.VectorSubcoreMesh(core_axis_name="c", subcore_axis_name="s")

    @functools.partial(
        pl.kernel,
        out_type=jax.ShapeDtypeStruct((NC, np_rows, d), jnp.float32),
        mesh=mesh,
        scratch_types=[
            pltpu.VMEM_SHARED((np_rows, d), jnp.float32),
            pltpu.VMEM((2, GRP, C), jnp.int32),
            pltpu.VMEM((2, GRP, C), jnp.int32),
            pltpu.VMEM((2, C, d), jnp.float32),
            pltpu.SemaphoreType.DMA,
            pltpu.SemaphoreType.DMA,
            pltpu.SemaphoreType.DMA,
        ],
    )
    def agg_kernel(g_hbm, src_hbm, dst_hbm, out_hbm,
                   acc, src_v, dst_v, rows_v, sem0, sem1, sem_idx):
        cid = lax.axis_index("c")
        sid = lax.axis_index("s")
        wid = cid * NS + sid
        _fill2d(rows_v.at[0], C, d, 0.0)
        for k in range(rows_per_tile // C):
            pltpu.sync_copy(rows_v.at[0],
                            acc.at[pl.ds(sid * rows_per_tile + k * C, C)])
        plsc.subcore_barrier()

        def wait_gather(buf, sem):
            pltpu.make_async_copy(g_hbm.at[pl.ds(0, C)], buf, sem).wait()

        ngroups = nch // GRP
        pltpu.sync_copy(src_hbm.at[wid, pl.ds(0, GRP)], src_v.at[0])
        pltpu.sync_copy(dst_hbm.at[wid, pl.ds(0, GRP)], dst_v.at[0])

        def group(gi, _):
            s = gi % 2
            sv = src_v.at[s]
            dv = dst_v.at[s]

            @pl.when(gi > 0)
            def _():
                pltpu.make_async_copy(src_hbm.at[wid, pl.ds(0, GRP)], sv,
                                      sem_idx).wait()
                pltpu.make_async_copy(dst_hbm.at[wid, pl.ds(0, GRP)], dv,
                                      sem_idx).wait()

            @pl.when(gi + 1 < ngroups)
            def _():
                pltpu.async_copy(
                    src_hbm.at[wid, pl.ds((gi + 1) * GRP, GRP)],
                    src_v.at[1 - s], sem_idx)
                pltpu.async_copy(
                    dst_hbm.at[wid, pl.ds((gi + 1) * GRP, GRP)],
                    dst_v.at[1 - s], sem_idx)

            pltpu.async_copy(g_hbm.at[sv.at[0]], rows_v.at[0], sem0)

            def pair(t, _):
                pltpu.async_copy(g_hbm.at[sv.at[2 * t + 1]], rows_v.at[1],
                                 sem1)
                wait_gather(rows_v.at[0], sem0)
                pltpu.sync_copy(rows_v.at[0], acc.at[dv.at[2 * t]],
                                add=True)

                @pl.when(t + 1 < GRP // 2)
                def _():
                    pltpu.async_copy(g_hbm.at[sv.at[2 * t + 2]],
                                     rows_v.at[0], sem0)
                wait_gather(rows_v.at[1], sem1)
                pltpu.sync_copy(rows_v.at[1], acc.at[dv.at[2 * t + 1]],
                                add=True)
                return 0
            lax.fori_loop(0, GRP // 2, pair, 0)
            return 0
        lax.fori_loop(0, ngroups, group, 0)
        plsc.subcore_barrier()
        pltpu.sync_copy(acc.at[pl.ds(sid * rows_per_tile, rows_per_tile)],
                        out_hbm.at[cid, pl.ds(sid * rows_per_tile, rows_per_tile)])

    return agg_kernel(g, src3d, dst3d)


def _tc_first(degp, xp, w0, np_rows, blk):
    din, dh = w0.shape

    def body(degp_ref, x_ref, w_ref, dinv_ref, g_ref):
        deg = degp_ref[0] + degp_ref[1] + 1.0
        dv = lax.rsqrt(deg)
        dinv_ref[...] = dv
        h = jnp.dot(x_ref[...], w_ref[...], preferred_element_type=jnp.float32)
        g_ref[...] = h * dv[:, None]

    grid = (np_rows // blk,)
    return pl.pallas_call(
        body,
        grid=grid,
        in_specs=[
            pl.BlockSpec((NC, blk), lambda i: (0, i)),
            pl.BlockSpec((blk, din), lambda i: (i, 0)),
            pl.BlockSpec((din, dh), lambda i: (0, 0)),
        ],
        out_specs=[
            pl.BlockSpec((blk,), lambda i: (i,)),
            pl.BlockSpec((blk, dh), lambda i: (i, 0)),
        ],
        out_shape=[
            jax.ShapeDtypeStruct((np_rows,), jnp.float32),
            jax.ShapeDtypeStruct((np_rows, dh), jnp.float32),
        ],
    )(degp, xp, w0)


def _tc_mid(aggp, g, dinv, b, w, np_rows, blk):
    d, dn = w.shape

    def body(aggp_ref, g_ref, dinv_ref, b_ref, w_ref, out_ref):
        s = aggp_ref[0] + aggp_ref[1] + g_ref[...]
        dv = dinv_ref[...]
        h = jnp.maximum(s * dv[:, None] + b_ref[...][None, :], 0.0)
        out_ref[...] = jnp.dot(h, w_ref[...],
                               preferred_element_type=jnp.float32) * dv[:, None]

    grid = (np_rows // blk,)
    return pl.pallas_call(
        body,
        grid=grid,
        in_specs=[
            pl.BlockSpec((NC, blk, d), lambda i: (0, i, 0)),
            pl.BlockSpec((blk, d), lambda i: (i, 0)),
            pl.BlockSpec((blk,), lambda i: (i,)),
            pl.BlockSpec((d,), lambda i: (0,)),
            pl.BlockSpec((d, dn), lambda i: (0, 0)),
        ],
        out_specs=pl.BlockSpec((blk, dn), lambda i: (i, 0)),
        out_shape=jax.ShapeDtypeStruct((np_rows, dn), jnp.float32),
    )(aggp, g, dinv, b, w)


def _tc_final(aggp, g, dinv, b, np_rows, blk):
    d = g.shape[1]
    dout = b.shape[0]

    def body(aggp_ref, g_ref, dinv_ref, b_ref, out_ref):
        s = aggp_ref[0] + aggp_ref[1] + g_ref[...]
        v = (s * dinv_ref[...][:, None])[:, :dout] + b_ref[...][None, :]
        m = jnp.max(v, axis=-1, keepdims=True)
        e = v - m
        out_ref[...] = e - jnp.log(jnp.sum(jnp.exp(e), axis=-1, keepdims=True))

    grid = (np_rows // blk,)
    return pl.pallas_call(
        body,
        grid=grid,
        in_specs=[
            pl.BlockSpec((NC, blk, d), lambda i: (0, i, 0)),
            pl.BlockSpec((blk, d), lambda i: (i, 0)),
            pl.BlockSpec((blk,), lambda i: (i,)),
            pl.BlockSpec((dout,), lambda i: (0,)),
        ],
        out_specs=pl.BlockSpec((blk, dout), lambda i: (i, 0)),
        out_shape=jax.ShapeDtypeStruct((np_rows, dout), jnp.float32),
    )(aggp, g, dinv, b)


def kernel(x, edge_index, W0, b0, W1, b1, W2, b2):
    n, din = x.shape
    e = edge_index.shape[1]

    np_rows = ((n + 16) + 16 * NW - 1) // (16 * NW) * (16 * NW)
    n_pad_rows = np_rows - n
    ept = (e + NW - 1) // NW
    nch = (ept + C * GRP - 1) // (C * GRP) * GRP
    e_pad = NW * nch * C
    pad = e_pad - e

    pad_ids = n + (jnp.arange(pad, dtype=jnp.int32) % n_pad_rows)
    src3d = jnp.concatenate([edge_index[0], pad_ids]).reshape(NW, nch, C)
    dst3d = jnp.concatenate([edge_index[1], pad_ids]).reshape(NW, nch, C)
    xp = jnp.pad(x, ((0, np_rows - n), (0, 0)))

    blk = 1024
    degp = _sc_degree(dst3d, np_rows, nch)
    dinv, g0 = _tc_first(degp, xp, W0, np_rows, blk)
    a0 = _sc_aggregate(g0, src3d, dst3d, np_rows, nch, W0.shape[1])
    g1 = _tc_mid(a0, g0, dinv, b0, W1, np_rows, blk)
    a1 = _sc_aggregate(g1, src3d, dst3d, np_rows, nch, W1.shape[1])
    W2p = jnp.pad(W2, ((0, 0), (0, 128 - W2.shape[1])))
    g2 = _tc_mid(a1, g1, dinv, b1, W2p, np_rows, blk)
    a2 = _sc_aggregate(g2, src3d, dst3d, np_rows, nch, W2p.shape[1])
    out = _tc_final(a2, g2, dinv, b2, np_rows, blk)
    return out[:n]
